# baseline (device time: 24661 ns/iter reference)
import os

import jax
import jax.numpy as jnp
from jax import lax
from jax.experimental import pallas as pl
from jax.experimental.pallas import tpu as pltpu

N_DEV = 32
DH = 64


def kernel(x, Wq, Wo, Wk, Wv):
    B, Sq, D = x.shape
    Hd = Wq.shape[1]
    Hq = Hd // DH
    R = B * Sq
    P = R // N_DEV
    bf16 = jnp.bfloat16
    comm_on = os.environ.get("ABLATE_COMM") != "1"
    floor = os.environ.get("ABLATE_FLOOR") == "1"

    def body(x_ref, wq_ref, wo_ref, wk_ref, wv_ref, out_ref,
             o_ref, acc_ref, s_ref, sbuf_ref, rs_ref, agbuf_ref,
             wq_v, wk_v, wv_v, wo_v,
             rs_send_sem, rs_recv_sem, ag_send_sem, ag_recv_sem, w_sems):
        my_pos = lax.axis_index("i")

        if floor:
            out_ref[:] = x_ref[:].astype(bf16)
            return

        w_cp = []
        for i, (src, dst) in enumerate(
                [(wq_ref, wq_v), (wk_ref, wk_v), (wv_ref, wv_v),
                 (wo_ref, wo_v)]):
            cp = pltpu.make_async_copy(src, dst, w_sems.at[i])
            cp.start()
            w_cp.append(cp)

        if comm_on:
            barrier_sem = pltpu.get_barrier_semaphore()
            for j in range(N_DEV):
                pl.semaphore_signal(
                    barrier_sem, inc=1, device_id=j,
                    device_id_type=pl.DeviceIdType.LOGICAL,
                )

        x2 = x_ref[:].reshape(R, D).astype(bf16)
        w_cp[0].wait()
        qf = jnp.dot(x2 * 0.125, wq_v[:].astype(bf16),
                     preferred_element_type=jnp.float32).astype(bf16)
        w_cp[1].wait()
        kf = jnp.dot(x2, wk_v[:].astype(bf16),
                     preferred_element_type=jnp.float32).astype(bf16)
        w_cp[2].wait()
        vf = jnp.dot(x2, wv_v[:].astype(bf16),
                     preferred_element_type=jnp.float32).astype(bf16)
        w_cp[3].wait()
        wo_b = wo_v[:].astype(bf16)

        def compute_batch(b):
            rows = slice(b * Sq, (b + 1) * Sq)
            for h in range(Hq):
                cols = slice(h * DH, (h + 1) * DH)
                s_ref[h] = lax.dot_general(
                    qf[rows, cols], kf[rows, cols], (((1,), (1,)), ((), ())),
                    preferred_element_type=jnp.float32,
                )
            s_all = s_ref[:]
            m = jnp.max(s_all, axis=-1, keepdims=True)
            p_ = jnp.exp(s_all - m)
            l_ = jnp.sum(p_, axis=-1, keepdims=True)
            pn = (p_ / l_).astype(bf16)
            for h in range(Hq):
                cols = slice(h * DH, (h + 1) * DH)
                o_ref[b, :, cols] = jnp.dot(
                    pn[h], vf[rows, cols], preferred_element_type=jnp.float32
                ).astype(bf16)
            acc_ref[b] = jnp.dot(
                o_ref[b], wo_b, preferred_element_type=jnp.float32,
            )

        def rs_send(j):
            return pltpu.make_async_remote_copy(
                src_ref=sbuf_ref.at[j],
                dst_ref=rs_ref.at[my_pos],
                send_sem=rs_send_sem,
                recv_sem=rs_recv_sem,
                device_id=j,
                device_id_type=pl.DeviceIdType.LOGICAL,
            )

        compute_batch(0)
        if comm_on:
            pl.semaphore_wait(barrier_sem, N_DEV)
            sbuf_ref[0:N_DEV // 2] = acc_ref[0].astype(bf16).reshape(
                N_DEV // 2, P, D)
            for j in range(N_DEV // 2):
                rs_send(j).start()
        compute_batch(1)

        if not comm_on:
            out_ref[:] = acc_ref[:].astype(bf16)
            return

        sbuf_ref[N_DEV // 2:N_DEV] = acc_ref[1].astype(bf16).reshape(
            N_DEV // 2, P, D)
        for j in range(N_DEV // 2, N_DEV):
            rs_send(j).start()

        for _ in range(N_DEV):
            rs_send(0).wait_recv()
        agbuf_ref[:] = jnp.sum(
            rs_ref[:].astype(jnp.float32), axis=0).astype(bf16)

        ppb = N_DEV // B
        b_idx = my_pos // ppb
        r0 = (my_pos % ppb) * P

        def ag_send(j):
            return pltpu.make_async_remote_copy(
                src_ref=agbuf_ref,
                dst_ref=out_ref.at[b_idx, pl.ds(r0, P)],
                send_sem=ag_send_sem,
                recv_sem=ag_recv_sem,
                device_id=j,
                device_id_type=pl.DeviceIdType.LOGICAL,
            )

        for j in range(N_DEV):
            ag_send(j).start()
        for _ in range(N_DEV):
            ag_send(0).wait_recv()

        for _ in range(N_DEV):
            rs_send(0).wait_send()
            ag_send(0).wait_send()

    return pl.pallas_call(
        body,
        out_shape=jax.ShapeDtypeStruct((B, Sq, D), bf16),
        in_specs=[
            pl.BlockSpec(memory_space=pltpu.VMEM),
            pl.BlockSpec(memory_space=pl.ANY),
            pl.BlockSpec(memory_space=pl.ANY),
            pl.BlockSpec(memory_space=pl.ANY),
            pl.BlockSpec(memory_space=pl.ANY),
        ],
        out_specs=pl.BlockSpec(memory_space=pltpu.VMEM),
        scratch_shapes=[
            pltpu.VMEM((B, Sq, Hd), bf16),
            pltpu.VMEM((B, Sq, D), jnp.float32),
            pltpu.VMEM((Hq, Sq, Sq), jnp.float32),
            pltpu.VMEM((N_DEV, P, D), bf16),
            pltpu.VMEM((N_DEV, P, D), bf16),
            pltpu.VMEM((P, D), bf16),
            pltpu.VMEM((D, Hd), jnp.float32),
            pltpu.VMEM((D, Hd), jnp.float32),
            pltpu.VMEM((D, Hd), jnp.float32),
            pltpu.VMEM((Hd, D), jnp.float32),
            pltpu.SemaphoreType.DMA,
            pltpu.SemaphoreType.DMA,
            pltpu.SemaphoreType.DMA,
            pltpu.SemaphoreType.DMA,
            pltpu.SemaphoreType.DMA((4,)),
        ],
        compiler_params=(
            pltpu.CompilerParams(collective_id=0) if comm_on
            else pltpu.CompilerParams()
        ),
    )(x, Wq, Wo, Wk, Wv)


# device time: 23521 ns/iter; 1.0485x vs baseline; 1.0485x over previous
import os

import jax
import jax.numpy as jnp
from jax import lax
from jax.experimental import pallas as pl
from jax.experimental.pallas import tpu as pltpu

N_DEV = 32
DH = 64


def kernel(x, Wq, Wo, Wk, Wv):
    B, Sq, D = x.shape
    Hd = Wq.shape[1]
    Hq = Hd // DH
    R = B * Sq
    P = R // N_DEV
    bf16 = jnp.bfloat16
    comm_on = os.environ.get("ABLATE_COMM") != "1"
    floor = os.environ.get("ABLATE_FLOOR") == "1"

    def body(x_ref, wq_ref, wo_ref, wk_ref, wv_ref, out_ref,
             o_ref, acc_ref, s_ref, sbuf_ref, rs_ref, agbuf_ref,
             rs_send_sem, rs_recv_sem, ag_send_sem, ag_recv_sem):
        my_pos = lax.axis_index("i")

        if floor:
            out_ref[:] = x_ref[:].astype(bf16)
            return


        if comm_on:
            barrier_sem = pltpu.get_barrier_semaphore()
            for j in range(N_DEV):
                pl.semaphore_signal(
                    barrier_sem, inc=1, device_id=j,
                    device_id_type=pl.DeviceIdType.LOGICAL,
                )

        x2 = x_ref[:].reshape(R, D).astype(bf16)
        qf = jnp.dot(x2 * 0.125, wq_ref[:].astype(bf16),
                     preferred_element_type=jnp.float32).astype(bf16)
        kf = jnp.dot(x2, wk_ref[:].astype(bf16),
                     preferred_element_type=jnp.float32).astype(bf16)
        vf = jnp.dot(x2, wv_ref[:].astype(bf16),
                     preferred_element_type=jnp.float32).astype(bf16)
        wo_b = wo_ref[:].astype(bf16)

        def compute_batch(b):
            rows = slice(b * Sq, (b + 1) * Sq)
            for h in range(Hq):
                cols = slice(h * DH, (h + 1) * DH)
                s_ref[h] = lax.dot_general(
                    qf[rows, cols], kf[rows, cols], (((1,), (1,)), ((), ())),
                    preferred_element_type=jnp.float32,
                )
            s_all = s_ref[:]
            m = jnp.max(s_all, axis=-1, keepdims=True)
            p_ = jnp.exp(s_all - m)
            l_ = jnp.sum(p_, axis=-1, keepdims=True)
            pn = (p_ / l_).astype(bf16)
            for h in range(Hq):
                cols = slice(h * DH, (h + 1) * DH)
                o_ref[b, :, cols] = jnp.dot(
                    pn[h], vf[rows, cols], preferred_element_type=jnp.float32
                ).astype(bf16)
            acc_ref[b] = jnp.dot(
                o_ref[b], wo_b, preferred_element_type=jnp.float32,
            )

        def rs_send(j):
            return pltpu.make_async_remote_copy(
                src_ref=sbuf_ref.at[j],
                dst_ref=rs_ref.at[my_pos],
                send_sem=rs_send_sem,
                recv_sem=rs_recv_sem,
                device_id=j,
                device_id_type=pl.DeviceIdType.LOGICAL,
            )

        compute_batch(0)
        if comm_on:
            pl.semaphore_wait(barrier_sem, N_DEV)
            sbuf_ref[0:N_DEV // 2] = acc_ref[0].astype(bf16).reshape(
                N_DEV // 2, P, D)
            for j in range(N_DEV // 2):
                rs_send(j).start()
        compute_batch(1)

        if not comm_on:
            out_ref[:] = acc_ref[:].astype(bf16)
            return

        sbuf_ref[N_DEV // 2:N_DEV] = acc_ref[1].astype(bf16).reshape(
            N_DEV // 2, P, D)
        for j in range(N_DEV // 2, N_DEV):
            rs_send(j).start()

        for _ in range(N_DEV):
            rs_send(0).wait_recv()
        agbuf_ref[:] = jnp.sum(
            rs_ref[:].astype(jnp.float32), axis=0).astype(bf16)

        ppb = N_DEV // B
        b_idx = my_pos // ppb
        r0 = (my_pos % ppb) * P

        def ag_send(j):
            return pltpu.make_async_remote_copy(
                src_ref=agbuf_ref,
                dst_ref=out_ref.at[b_idx, pl.ds(r0, P)],
                send_sem=ag_send_sem,
                recv_sem=ag_recv_sem,
                device_id=j,
                device_id_type=pl.DeviceIdType.LOGICAL,
            )

        for j in range(N_DEV):
            ag_send(j).start()
        for _ in range(N_DEV):
            ag_send(0).wait_recv()

        for _ in range(N_DEV):
            rs_send(0).wait_send()
            ag_send(0).wait_send()

    return pl.pallas_call(
        body,
        out_shape=jax.ShapeDtypeStruct((B, Sq, D), bf16),
        in_specs=[pl.BlockSpec(memory_space=pltpu.VMEM)] * 5,
        out_specs=pl.BlockSpec(memory_space=pltpu.VMEM),
        scratch_shapes=[
            pltpu.VMEM((B, Sq, Hd), bf16),
            pltpu.VMEM((B, Sq, D), jnp.float32),
            pltpu.VMEM((Hq, Sq, Sq), jnp.float32),
            pltpu.VMEM((N_DEV, P, D), bf16),
            pltpu.VMEM((N_DEV, P, D), bf16),
            pltpu.VMEM((P, D), bf16),
            pltpu.SemaphoreType.DMA,
            pltpu.SemaphoreType.DMA,
            pltpu.SemaphoreType.DMA,
            pltpu.SemaphoreType.DMA,
        ],
        compiler_params=(
            pltpu.CompilerParams(collective_id=0) if comm_on
            else pltpu.CompilerParams()
        ),
    )(x, Wq, Wo, Wk, Wv)
